# Initial kernel scaffold; baseline (speedup 1.0000x reference)
import jax
import jax.numpy as jnp
from jax.experimental import pallas as pl


def kernel(x_nchw, gamma, beta):
    raise NotImplementedError("write your pallas kernel here")


# speedup vs baseline: 1.6013x; 1.6013x over previous
import functools

import jax
import jax.numpy as jnp
from jax.experimental import pallas as pl
from jax.experimental.pallas import tpu as pltpu

_EPS = 1e-5
_VMEM_LIMIT = 48 * 1024 * 1024


def _row_stats_kernel(x_ref, st_ref):
    x = x_ref[...]
    s = jnp.sum(x, axis=(1, 2))
    ss = jnp.sum(x * x, axis=(1, 2))
    st_ref[...] = jnp.stack([s, ss], axis=1)


def _silu_apply_kernel(x_ref, ss_ref, o_ref):
    x = x_ref[...]
    scale = ss_ref[:, 0:1][..., None]
    shift = ss_ref[:, 1:2][..., None]
    z = x * scale + shift
    e = jnp.exp(-jnp.maximum(z, -80.0))
    d = 1.0 + e
    r = pl.reciprocal(d, approx=True)
    r = r * (2.0 - d * r)
    o_ref[...] = z * r


def kernel(x_nchw, gamma, beta):
    N, C, H, W = x_nchw.shape
    R = N * C
    cnt = N * H * W

    x3 = x_nchw.reshape(R, H, W)

    r_tile = C if R % C == 0 else 8
    grid = (R // r_tile,)

    stats = pl.pallas_call(
        _row_stats_kernel,
        out_shape=jax.ShapeDtypeStruct((R, 2), jnp.float32),
        grid=grid,
        in_specs=[pl.BlockSpec((r_tile, H, W), lambda r: (r, 0, 0))],
        out_specs=pl.BlockSpec((r_tile, 2), lambda r: (r, 0)),
        compiler_params=pltpu.CompilerParams(
            dimension_semantics=("parallel",),
            vmem_limit_bytes=_VMEM_LIMIT),
    )(x3)

    st = stats.reshape(N, C, 2)
    sum_c = jnp.sum(st[:, :, 0], axis=0)
    ssq_c = jnp.sum(st[:, :, 1], axis=0)
    mean_c = sum_c / cnt
    var_c = ssq_c / cnt - mean_c * mean_c
    inv_std = jax.lax.rsqrt(var_c + _EPS)
    scale_c = gamma.astype(jnp.float32) * inv_std
    shift_c = beta.astype(jnp.float32) - mean_c * scale_c
    ss_rows = jnp.stack(
        [jnp.broadcast_to(scale_c[None, :], (N, C)).reshape(R),
         jnp.broadcast_to(shift_c[None, :], (N, C)).reshape(R)], axis=1)

    out3 = pl.pallas_call(
        _silu_apply_kernel,
        out_shape=jax.ShapeDtypeStruct((R, H, W), jnp.float32),
        grid=grid,
        in_specs=[pl.BlockSpec((r_tile, H, W), lambda r: (r, 0, 0)),
                  pl.BlockSpec((r_tile, 2), lambda r: (r, 0))],
        out_specs=pl.BlockSpec((r_tile, H, W), lambda r: (r, 0, 0)),
        compiler_params=pltpu.CompilerParams(
            dimension_semantics=("parallel",),
            vmem_limit_bytes=_VMEM_LIMIT),
    )(x3, ss_rows)

    return out3.reshape(N, C, H, W)
